# baseline (speedup 1.0000x reference)
import functools

import jax
import jax.numpy as jnp
from jax import lax
from jax.experimental import pallas as pl
from jax.experimental.pallas import tpu as pltpu
from jax.experimental.pallas import tpu_sc as plsc

_ROWS = 257
_D = 128
_NW = 16
_RPW = 256 // _NW

_mesh = plsc.VectorSubcoreMesh(core_axis_name="c", subcore_axis_name="s", num_cores=1)


@functools.partial(
    pl.kernel,
    mesh=_mesh,
    out_type=jax.ShapeDtypeStruct((_ROWS, _D), jnp.float32),
    scratch_types=[pltpu.VMEM((_RPW + 1, _D), jnp.float32)],
)
def _lookup(table_hbm, out_hbm, buf):
    wid = lax.axis_index("s")
    base = wid * _RPW
    pltpu.sync_copy(table_hbm.at[pl.ds(base, _RPW)], buf.at[pl.ds(0, _RPW)])
    pltpu.sync_copy(buf.at[pl.ds(0, _RPW)], out_hbm.at[pl.ds(base, _RPW)])

    @pl.when(wid == 0)
    def _tail():
        pltpu.sync_copy(table_hbm.at[pl.ds(256, 1)], buf.at[pl.ds(_RPW, 1)])
        pltpu.sync_copy(buf.at[pl.ds(_RPW, 1)], out_hbm.at[pl.ds(256, 1)])


def kernel(seq_len, table):
    del seq_len
    return _lookup(table)


# speedup vs baseline: 1.0258x; 1.0258x over previous
import functools

import jax
import jax.numpy as jnp
from jax import lax
from jax.experimental import pallas as pl
from jax.experimental.pallas import tpu as pltpu
from jax.experimental.pallas import tpu_sc as plsc

_ROWS = 257
_D = 128
_RPW = 16
_H = 8

_mesh = plsc.VectorSubcoreMesh(core_axis_name="c", subcore_axis_name="s", num_cores=1)


@functools.partial(
    pl.kernel,
    mesh=_mesh,
    out_type=jax.ShapeDtypeStruct((_ROWS, _D), jnp.float32),
    scratch_types=[
        pltpu.VMEM((_RPW + 1, _D), jnp.float32),
        pltpu.SemaphoreType.DMA,
        pltpu.SemaphoreType.DMA,
        pltpu.SemaphoreType.DMA,
        pltpu.SemaphoreType.DMA,
        pltpu.SemaphoreType.DMA,
        pltpu.SemaphoreType.DMA,
    ],
)
def _lookup(table_hbm, out_hbm, buf, s1, s2, s3, s4, s5, s6):
    wid = lax.axis_index("s")
    base = wid * _RPW

    def _halves():
        in0 = pltpu.async_copy(
            table_hbm.at[pl.ds(base, _H)], buf.at[pl.ds(0, _H)], s1)
        in1 = pltpu.async_copy(
            table_hbm.at[pl.ds(base + _H, _H)], buf.at[pl.ds(_H, _H)], s2)
        in0.wait()
        out0 = pltpu.async_copy(
            buf.at[pl.ds(0, _H)], out_hbm.at[pl.ds(base, _H)], s3)
        in1.wait()
        out1 = pltpu.async_copy(
            buf.at[pl.ds(_H, _H)], out_hbm.at[pl.ds(base + _H, _H)], s4)
        return out0, out1

    @pl.when(wid == 0)
    def _tile0():
        tin = pltpu.async_copy(
            table_hbm.at[pl.ds(256, 1)], buf.at[pl.ds(_RPW, 1)], s5)
        out0, out1 = _halves()
        tin.wait()
        tout = pltpu.async_copy(
            buf.at[pl.ds(_RPW, 1)], out_hbm.at[pl.ds(256, 1)], s6)
        out0.wait()
        out1.wait()
        tout.wait()

    @pl.when(wid > 0)
    def _rest():
        out0, out1 = _halves()
        out0.wait()
        out1.wait()


def kernel(seq_len, table):
    del seq_len
    return _lookup(table)
